# baseline (speedup 1.0000x reference)
import jax
import jax.numpy as jnp
from jax.experimental import pallas as pl

_HEIGHT = 4096
_WIDTH = 4096
_BATCH = 2048
_NBLK = 512
_NSUB = 128
_MBLK = 1024


def _decode_matmul_kernel(x_ref, code_ref, mm_ref, out_ref):
    zeros = jnp.zeros((_NSUB, _WIDTH), jnp.bfloat16)
    for nb in range(0, _NBLK, _NSUB):
        c = code_ref[nb:nb + _NSUB, :].astype(jnp.bfloat16)
        mins_b = jnp.broadcast_to(mm_ref[nb:nb + _NSUB, 0:1], (_NSUB, _WIDTH))
        maxs_b = jnp.broadcast_to(mm_ref[nb:nb + _NSUB, 1:2], (_NSUB, _WIDTH))
        w = jnp.where(c == 1.0, mins_b, jnp.where(c == 2.0, maxs_b, zeros))
        for mb in range(0, _BATCH, _MBLK):
            out_ref[mb:mb + _MBLK, nb:nb + _NSUB] = jax.lax.dot_general(
                x_ref[mb:mb + _MBLK, :],
                w,
                (((1,), (1,)), ((), ())),
                preferred_element_type=jnp.float32,
            ).astype(jnp.bfloat16)


def kernel(x, w_tern, ter_minmax):
    mm = ter_minmax.reshape(_HEIGHT, 2)
    nj = _HEIGHT // _NBLK
    return pl.pallas_call(
        _decode_matmul_kernel,
        grid=(nj,),
        in_specs=[
            pl.BlockSpec((_BATCH, _WIDTH), lambda j: (0, 0)),
            pl.BlockSpec((_NBLK, _WIDTH), lambda j: (j, 0)),
            pl.BlockSpec((_NBLK, 2), lambda j: (j, 0)),
        ],
        out_specs=pl.BlockSpec((_BATCH, _NBLK), lambda j: (0, j)),
        out_shape=jax.ShapeDtypeStruct((_BATCH, _HEIGHT), jnp.bfloat16),
    )(x, w_tern, mm)


# speedup vs baseline: 1.7494x; 1.7494x over previous
import jax
import jax.numpy as jnp
from jax.experimental import pallas as pl

_HEIGHT = 4096
_WIDTH = 4096
_BATCH = 2048
_NBLK = 512
_NSUB = 256
_MBLK = 512


def _decode_matmul_kernel(x_ref, code_ref, mm_ref, out_ref):
    zeros = jnp.zeros((_NSUB, _WIDTH), jnp.bfloat16)
    for nb in range(0, _NBLK, _NSUB):
        c = code_ref[nb:nb + _NSUB, :].astype(jnp.bfloat16)
        mins_b = jnp.broadcast_to(mm_ref[nb:nb + _NSUB, 0:1], (_NSUB, _WIDTH))
        maxs_b = jnp.broadcast_to(mm_ref[nb:nb + _NSUB, 1:2], (_NSUB, _WIDTH))
        w = jnp.where(c == 1.0, mins_b, jnp.where(c == 2.0, maxs_b, zeros))
        for mb in range(0, _BATCH, _MBLK):
            out_ref[mb:mb + _MBLK, nb:nb + _NSUB] = jax.lax.dot_general(
                x_ref[mb:mb + _MBLK, :],
                w,
                (((1,), (1,)), ((), ())),
                preferred_element_type=jnp.float32,
            ).astype(jnp.bfloat16)


def kernel(x, w_tern, ter_minmax):
    mm = ter_minmax.reshape(_HEIGHT, 2)
    nj = _HEIGHT // _NBLK
    return pl.pallas_call(
        _decode_matmul_kernel,
        grid=(nj,),
        in_specs=[
            pl.BlockSpec((_BATCH, _WIDTH), lambda j: (0, 0)),
            pl.BlockSpec((_NBLK, _WIDTH), lambda j: (j, 0)),
            pl.BlockSpec((_NBLK, 2), lambda j: (j, 0)),
        ],
        out_specs=pl.BlockSpec((_BATCH, _NBLK), lambda j: (0, j)),
        out_shape=jax.ShapeDtypeStruct((_BATCH, _HEIGHT), jnp.bfloat16),
    )(x, w_tern, mm)


# speedup vs baseline: 1.7520x; 1.0015x over previous
import jax
import jax.numpy as jnp
from jax.experimental import pallas as pl
from jax.experimental.pallas import tpu as pltpu

_HEIGHT = 4096
_WIDTH = 4096
_BATCH = 2048
_NBLK = 512
_NSUB = 256
_MBLK = 1024


def _decode_matmul_kernel(x_ref, code_ref, mm_ref, out_ref, w0_ref, w1_ref):
    zeros = jnp.zeros((_NSUB, _WIDTH), jnp.bfloat16)
    w_refs = (w0_ref, w1_ref)
    for i, nb in enumerate(range(0, _NBLK, _NSUB)):
        w_ref = w_refs[i % 2]
        c = code_ref[nb:nb + _NSUB, :].astype(jnp.bfloat16)
        mins_b = jnp.broadcast_to(mm_ref[nb:nb + _NSUB, 0:1], (_NSUB, _WIDTH))
        maxs_b = jnp.broadcast_to(mm_ref[nb:nb + _NSUB, 1:2], (_NSUB, _WIDTH))
        w_ref[...] = jnp.where(c == 1.0, mins_b, jnp.where(c == 2.0, maxs_b,
                                                           zeros))
        for mb in range(0, _BATCH, _MBLK):
            out_ref[mb:mb + _MBLK, nb:nb + _NSUB] = jax.lax.dot_general(
                x_ref[mb:mb + _MBLK, :],
                w_ref[...],
                (((1,), (1,)), ((), ())),
                preferred_element_type=jnp.float32,
            ).astype(jnp.bfloat16)


def kernel(x, w_tern, ter_minmax):
    mm = ter_minmax.reshape(_HEIGHT, 2)
    nj = _HEIGHT // _NBLK
    return pl.pallas_call(
        _decode_matmul_kernel,
        grid=(nj,),
        in_specs=[
            pl.BlockSpec((_BATCH, _WIDTH), lambda j: (0, 0)),
            pl.BlockSpec((_NBLK, _WIDTH), lambda j: (j, 0)),
            pl.BlockSpec((_NBLK, 2), lambda j: (j, 0)),
        ],
        out_specs=pl.BlockSpec((_BATCH, _NBLK), lambda j: (0, j)),
        out_shape=jax.ShapeDtypeStruct((_BATCH, _HEIGHT), jnp.bfloat16),
        scratch_shapes=[pltpu.VMEM((_NSUB, _WIDTH), jnp.bfloat16),
                        pltpu.VMEM((_NSUB, _WIDTH), jnp.bfloat16)],
    )(x, w_tern, mm)


# speedup vs baseline: 1.7588x; 1.0039x over previous
import jax
import jax.numpy as jnp
from jax.experimental import pallas as pl

_HEIGHT = 4096
_WIDTH = 4096
_BATCH = 2048
_NBLK = 512
_NSUB = 256
_MBLK = 1024


def _decode_matmul_kernel(x_ref, code_ref, mm_ref, out_ref):
    zeros = jnp.zeros((_NSUB, _WIDTH), jnp.bfloat16)
    for nb in range(0, _NBLK, _NSUB):
        c = code_ref[nb:nb + _NSUB, :].astype(jnp.bfloat16)
        mins_b = jnp.broadcast_to(mm_ref[nb:nb + _NSUB, 0:1], (_NSUB, _WIDTH))
        maxs_b = jnp.broadcast_to(mm_ref[nb:nb + _NSUB, 1:2], (_NSUB, _WIDTH))
        w = jnp.where(c == 1.0, mins_b, jnp.where(c == 2.0, maxs_b, zeros))
        for mb in range(0, _BATCH, _MBLK):
            out_ref[mb:mb + _MBLK, nb:nb + _NSUB] = jax.lax.dot_general(
                x_ref[mb:mb + _MBLK, :],
                w,
                (((1,), (1,)), ((), ())),
                preferred_element_type=jnp.float32,
            ).astype(jnp.bfloat16)


def kernel(x, w_tern, ter_minmax):
    mm = ter_minmax.reshape(_HEIGHT, 2)
    nj = _HEIGHT // _NBLK
    return pl.pallas_call(
        _decode_matmul_kernel,
        grid=(nj,),
        in_specs=[
            pl.BlockSpec((_BATCH, _WIDTH), lambda j: (0, 0)),
            pl.BlockSpec((_NBLK, _WIDTH), lambda j: (j, 0)),
            pl.BlockSpec((_NBLK, 2), lambda j: (j, 0)),
        ],
        out_specs=pl.BlockSpec((_BATCH, _NBLK), lambda j: (0, j)),
        out_shape=jax.ShapeDtypeStruct((_BATCH, _HEIGHT), jnp.bfloat16),
    )(x, w_tern, mm)
